# baseline (speedup 1.0000x reference)
import functools
import math

import jax
import jax.numpy as jnp
from jax import lax
from jax.experimental import pallas as pl
from jax.experimental.pallas import tpu as pltpu
from jax.experimental.pallas import tpu_sc as plsc

D_MODEL = 64
SCALE = math.sqrt(D_MODEL)
NUM_CORES = 2
NUM_SUBCORES = 16
NUM_WORKERS = NUM_CORES * NUM_SUBCORES
CHUNK = 512


def _make_lookup(batch: int):
    b_per_w = batch // NUM_WORKERS
    n_chunks = b_per_w // CHUNK
    mesh = plsc.VectorSubcoreMesh(core_axis_name="c", subcore_axis_name="s")

    @functools.partial(
        pl.kernel,
        mesh=mesh,
        compiler_params=pltpu.CompilerParams(use_tc_tiling_on_sc=False),
        out_type=jax.ShapeDtypeStruct((batch, D_MODEL), jnp.float32),
        scratch_types=[
            pltpu.VMEM((CHUNK,), jnp.int32),
            pltpu.VMEM((CHUNK, D_MODEL), jnp.float32),
            pltpu.SemaphoreType.DMA,
        ],
    )
    def lookup(lut_hbm, idx_hbm, out_hbm, idx_v, rows_v, sem):
        wid = lax.axis_index("s") * NUM_CORES + lax.axis_index("c")
        base = wid * b_per_w

        def chunk_body(i, carry):
            off = pl.multiple_of(base + i * CHUNK, 8)
            pltpu.sync_copy(idx_hbm.at[pl.ds(off, CHUNK)], idx_v)
            pltpu.async_copy(lut_hbm.at[idx_v], rows_v, sem).wait()

            def scale_row(r, c):
                for j in range(D_MODEL // 16):
                    sl = pl.ds(j * 16, 16)
                    rows_v[r, sl] = rows_v[r, sl] * SCALE
                return c

            lax.fori_loop(0, CHUNK, scale_row, 0)
            pltpu.sync_copy(rows_v, out_hbm.at[pl.ds(off, CHUNK)])
            return carry

        lax.fori_loop(0, n_chunks, chunk_body, 0)

    return lookup


def kernel(x, lut):
    batch = x.shape[0] * x.shape[1]
    flat_idx = x.reshape(batch)
    out = _make_lookup(batch)(lut, flat_idx)
    return out.reshape(x.shape[0], x.shape[1], D_MODEL)


# speedup vs baseline: 1.1344x; 1.1344x over previous
import functools
import math

import jax
import jax.numpy as jnp
from jax import lax
from jax.experimental import pallas as pl
from jax.experimental.pallas import tpu as pltpu
from jax.experimental.pallas import tpu_sc as plsc

D_MODEL = 64
SCALE = math.sqrt(D_MODEL)
NUM_CORES = 2
NUM_SUBCORES = 16
NUM_WORKERS = NUM_CORES * NUM_SUBCORES
CHUNK = 400
ROWS_PER_ITER = 4


def _make_lookup(batch: int):
    b_per_w = batch // NUM_WORKERS
    n_chunks = b_per_w // CHUNK
    n_pairs = n_chunks // 2
    mesh = plsc.VectorSubcoreMesh(core_axis_name="c", subcore_axis_name="s")

    @functools.partial(
        pl.kernel,
        mesh=mesh,
        compiler_params=pltpu.CompilerParams(use_tc_tiling_on_sc=False),
        out_type=jax.ShapeDtypeStruct((batch, D_MODEL), jnp.float32),
        scratch_types=[
            pltpu.VMEM((b_per_w,), jnp.int32),
            pltpu.VMEM((CHUNK, D_MODEL), jnp.float32),
            pltpu.VMEM((CHUNK, D_MODEL), jnp.float32),
            pltpu.VMEM((CHUNK, D_MODEL), jnp.float32),
            pltpu.VMEM((CHUNK, D_MODEL), jnp.float32),
            pltpu.SemaphoreType.DMA,
            pltpu.SemaphoreType.DMA,
            pltpu.SemaphoreType.DMA,
            pltpu.SemaphoreType.DMA,
        ],
    )
    def lookup(lut_hbm, idx_hbm, out_hbm, idx_all, g0, g1, w0, w1,
               sg0, sg1, sw0, sw1):
        wid = lax.axis_index("s") * NUM_CORES + lax.axis_index("c")
        base = pl.multiple_of(wid * b_per_w, 8)
        pltpu.sync_copy(idx_hbm.at[pl.ds(base, b_per_w)], idx_all)

        gbufs, wbufs = (g0, g1), (w0, w1)
        sgs, sws = (sg0, sg1), (sw0, sw1)

        def idx_slice(c):
            return idx_all.at[pl.ds(pl.multiple_of(c * CHUNK, 8), CHUNK)]

        def start_gather(par, c):
            pltpu.async_copy(lut_hbm.at[idx_slice(c)], gbufs[par], sgs[par])

        def scale(par):
            gb, wb = gbufs[par], wbufs[par]

            def body(i, carry):
                r0 = i * ROWS_PER_ITER
                for dr in range(ROWS_PER_ITER):
                    r = r0 + dr
                    for j in range(D_MODEL // 16):
                        sl = pl.ds(j * 16, 16)
                        wb[r, sl] = gb[r, sl] * SCALE
                return carry

            lax.fori_loop(0, CHUNK // ROWS_PER_ITER, body, 0)

        def consume(par, c, wait_write, prefetch):
            pltpu.make_async_copy(
                lut_hbm.at[idx_slice(c)], gbufs[par], sgs[par]).wait()
            if wait_write:
                pltpu.make_async_copy(
                    wbufs[par], out_hbm.at[pl.ds(base, CHUNK)],
                    sws[par]).wait()
            scale(par)
            off = pl.multiple_of(base + c * CHUNK, 8)
            pltpu.async_copy(wbufs[par], out_hbm.at[pl.ds(off, CHUNK)],
                             sws[par])
            if prefetch:
                start_gather(par, c + 2)

        start_gather(0, 0)
        start_gather(1, 1)
        consume(0, 0, wait_write=False, prefetch=True)
        consume(1, 1, wait_write=False, prefetch=True)

        def pair_body(o, carry):
            consume(0, 2 * o, wait_write=True, prefetch=True)
            consume(1, 2 * o + 1, wait_write=True, prefetch=True)
            return carry

        lax.fori_loop(1, n_pairs - 1, pair_body, 0)

        c_last = n_chunks - 2
        consume(0, c_last, wait_write=True, prefetch=False)
        consume(1, c_last + 1, wait_write=True, prefetch=False)
        for par in (0, 1):
            pltpu.make_async_copy(
                wbufs[par], out_hbm.at[pl.ds(base, CHUNK)], sws[par]).wait()

    return lookup


def kernel(x, lut):
    batch = x.shape[0] * x.shape[1]
    flat_idx = x.reshape(batch)
    out = _make_lookup(batch)(lut, flat_idx)
    return out.reshape(x.shape[0], x.shape[1], D_MODEL)
